# baseline (speedup 1.0000x reference)
import functools

import jax
import jax.numpy as jnp
from jax.experimental import pallas as pl
from jax.experimental.pallas import tpu as pltpu


def _gate_kernel(inv_hw, x_ref, w1_ref, b1_ref, w2_ref, b2_ref, o_ref):
    x = x_ref[...]
    s = jnp.sum(x, axis=(2, 3))
    m = jnp.max(x, axis=(2, 3))
    pooled = jnp.concatenate([s * inv_hw, m], axis=0)
    hidden = jnp.maximum(
        jnp.dot(pooled, w1_ref[...],
                preferred_element_type=jnp.float32) + b1_ref[...], 0.0)
    att = jnp.dot(hidden, w2_ref[...],
                  preferred_element_type=jnp.float32) + b2_ref[...]
    scale = jax.nn.sigmoid(att[0:1, :] + att[1:2, :])
    o_ref[...] = x * scale.reshape(1, x.shape[1], 1, 1)


def kernel(x, w1, b1, w2, b2):
    B, C, H, W = x.shape
    R = w1.shape[1]

    b1r = b1.reshape(1, R)
    b2r = b2.reshape(1, C)

    return pl.pallas_call(
        functools.partial(_gate_kernel, 1.0 / float(H * W)),
        out_shape=jax.ShapeDtypeStruct((B, C, H, W), x.dtype),
        grid=(B,),
        in_specs=[pl.BlockSpec((1, C, H, W), lambda b: (b, 0, 0, 0)),
                  pl.BlockSpec((C, R), lambda b: (0, 0)),
                  pl.BlockSpec((1, R), lambda b: (0, 0)),
                  pl.BlockSpec((R, C), lambda b: (0, 0)),
                  pl.BlockSpec((1, C), lambda b: (0, 0))],
        out_specs=pl.BlockSpec((1, C, H, W), lambda b: (b, 0, 0, 0)),
        compiler_params=pltpu.CompilerParams(
            dimension_semantics=("parallel",)),
    )(x, w1, b1r, w2, b2r)
